# Initial kernel scaffold; baseline (speedup 1.0000x reference)
import jax
import jax.numpy as jnp
from jax.experimental import pallas as pl


def kernel(x, adj, W1, W2, W3, W4, W5, W6):
    raise NotImplementedError("write your pallas kernel here")


# speedup vs baseline: 1.1677x; 1.1677x over previous
import jax
import jax.numpy as jnp
from jax.experimental import pallas as pl
from jax.experimental.pallas import tpu as pltpu


_N = 4096
_F = 128
_BM = 512
_NB = _N // _BM


def _encdec_kernel(adj_ref, x_ref, w1_ref, w2_ref, w3_ref, w4_ref, w5_ref,
                   w6_ref, zig_ref, zhat_ref, feat_ref, sup_ref):
    def layer(src_ref, w_ref, active, dst_ref):
        s = src_ref[...] @ w_ref[...]
        if active:
            s = jnp.tanh(s)
        sup_ref[...] = s.astype(jnp.bfloat16)

        def body(i, _):
            rows = pl.ds(i * _BM, _BM)
            dst_ref[rows, :] = jax.lax.dot_general(
                adj_ref[rows, :], sup_ref[...],
                (((1,), (0,)), ((), ())),
                preferred_element_type=jnp.float32)
            return 0

        jax.lax.fori_loop(0, _NB, body, 0)

    layer(x_ref, w1_ref, True, feat_ref)
    layer(feat_ref, w2_ref, True, feat_ref)
    layer(feat_ref, w3_ref, False, zig_ref)
    layer(zig_ref, w4_ref, True, feat_ref)
    layer(feat_ref, w5_ref, True, feat_ref)
    layer(feat_ref, w6_ref, True, zhat_ref)


def _gram_kernel(z_ref, zfull_ref, out_ref):
    zb = z_ref[...].astype(jnp.bfloat16)
    zf = zfull_ref[...].astype(jnp.bfloat16)
    s = jax.lax.dot_general(
        zb, zf, (((1,), (1,)), ((), ())),
        preferred_element_type=jnp.float32)
    out_ref[...] = jax.nn.sigmoid(s)


def _gram(z, block_rows=512):
    n, f = z.shape
    return pl.pallas_call(
        _gram_kernel,
        grid=(n // block_rows,),
        in_specs=[
            pl.BlockSpec((block_rows, f), lambda i: (i, 0)),
            pl.BlockSpec((n, f), lambda i: (0, 0)),
        ],
        out_specs=pl.BlockSpec((block_rows, n), lambda i: (i, 0)),
        out_shape=jax.ShapeDtypeStruct((n, n), jnp.float32),
    )(z, z)


def _pad_w(w):
    fin, fout = w.shape
    return jnp.pad(w, ((0, _F - fin), (0, _F - fout)))


def kernel(x, adj, W1, W2, W3, W4, W5, W6):
    nz = W3.shape[1]
    adj16 = adj.astype(jnp.bfloat16)
    ws = [_pad_w(w) for w in (W1, W2, W3, W4, W5, W6)]
    zig_pad, z_hat = pl.pallas_call(
        _encdec_kernel,
        out_shape=[
            jax.ShapeDtypeStruct((_N, _F), jnp.float32),
            jax.ShapeDtypeStruct((_N, _F), jnp.float32),
        ],
        scratch_shapes=[
            pltpu.VMEM((_N, _F), jnp.float32),
            pltpu.VMEM((_N, _F), jnp.bfloat16),
        ],
    )(adj16, x, *ws)
    z_igae = zig_pad[:, :nz]
    z_igae_adj = _gram(zig_pad)
    z_hat_adj = _gram(z_hat)
    return (z_igae, z_igae_adj, z_hat, z_hat_adj)


# speedup vs baseline: 1.2292x; 1.0527x over previous
import jax
import jax.numpy as jnp
from jax.experimental import pallas as pl
from jax.experimental.pallas import tpu as pltpu


_N = 4096
_F = 128
_BM = 512
_NB = _N // _BM


def _layer1_kernel(adj_ref, x_ref, w1_ref, out_ref, adj16_ref, sup_ref):
    @pl.when(pl.program_id(0) == 0)
    def _():
        sup_ref[...] = jnp.tanh(x_ref[...] @ w1_ref[...]).astype(jnp.bfloat16)

    a = adj_ref[...].astype(jnp.bfloat16)
    adj16_ref[...] = a
    out_ref[...] = jax.lax.dot_general(
        a, sup_ref[...], (((1,), (0,)), ((), ())),
        preferred_element_type=jnp.float32)


def _layers26_kernel(adj_ref, z1_ref, w2_ref, w3_ref, w4_ref, w5_ref,
                     w6_ref, zig_ref, zhat_ref, feat_ref, sup_ref):
    def layer(src_ref, w_ref, active, dst_ref):
        s = src_ref[...] @ w_ref[...]
        if active:
            s = jnp.tanh(s)
        sup_ref[...] = s.astype(jnp.bfloat16)

        def body(i, _):
            rows = pl.ds(i * _BM, _BM)
            dst_ref[rows, :] = jax.lax.dot_general(
                adj_ref[rows, :], sup_ref[...],
                (((1,), (0,)), ((), ())),
                preferred_element_type=jnp.float32)
            return 0

        jax.lax.fori_loop(0, _NB, body, 0)

    layer(z1_ref, w2_ref, True, feat_ref)
    layer(feat_ref, w3_ref, False, zig_ref)
    layer(zig_ref, w4_ref, True, feat_ref)
    layer(feat_ref, w5_ref, True, feat_ref)
    layer(feat_ref, w6_ref, True, zhat_ref)


def _gram_kernel(z_ref, zfull_ref, out_ref):
    zb = z_ref[...].astype(jnp.bfloat16)
    zf = zfull_ref[...].astype(jnp.bfloat16)
    s = jax.lax.dot_general(
        zb, zf, (((1,), (1,)), ((), ())),
        preferred_element_type=jnp.float32)
    out_ref[...] = jax.nn.sigmoid(s)


def _gram(z, block_rows=512):
    n, f = z.shape
    return pl.pallas_call(
        _gram_kernel,
        grid=(n // block_rows,),
        in_specs=[
            pl.BlockSpec((block_rows, f), lambda i: (i, 0)),
            pl.BlockSpec((n, f), lambda i: (0, 0)),
        ],
        out_specs=pl.BlockSpec((block_rows, n), lambda i: (i, 0)),
        out_shape=jax.ShapeDtypeStruct((n, n), jnp.float32),
    )(z, z)


def _pad_w(w):
    fin, fout = w.shape
    return jnp.pad(w, ((0, _F - fin), (0, _F - fout)))


def kernel(x, adj, W1, W2, W3, W4, W5, W6):
    nz = W3.shape[1]
    w1, w2, w3, w4, w5, w6 = (_pad_w(w) for w in (W1, W2, W3, W4, W5, W6))
    z1, adj16 = pl.pallas_call(
        _layer1_kernel,
        grid=(_NB,),
        in_specs=[
            pl.BlockSpec((_BM, _N), lambda i: (i, 0)),
            pl.BlockSpec((_N, _F), lambda i: (0, 0)),
            pl.BlockSpec((_F, _F), lambda i: (0, 0)),
        ],
        out_specs=[
            pl.BlockSpec((_BM, _F), lambda i: (i, 0)),
            pl.BlockSpec((_BM, _N), lambda i: (i, 0)),
        ],
        out_shape=[
            jax.ShapeDtypeStruct((_N, _F), jnp.float32),
            jax.ShapeDtypeStruct((_N, _N), jnp.bfloat16),
        ],
        scratch_shapes=[pltpu.VMEM((_N, _F), jnp.bfloat16)],
    )(adj, x, w1)
    zig_pad, z_hat = pl.pallas_call(
        _layers26_kernel,
        out_shape=[
            jax.ShapeDtypeStruct((_N, _F), jnp.float32),
            jax.ShapeDtypeStruct((_N, _F), jnp.float32),
        ],
        scratch_shapes=[
            pltpu.VMEM((_N, _F), jnp.float32),
            pltpu.VMEM((_N, _F), jnp.bfloat16),
        ],
    )(adj16, z1, w2, w3, w4, w5, w6)
    z_igae = zig_pad[:, :nz]
    z_igae_adj = _gram(zig_pad)
    z_hat_adj = _gram(z_hat)
    return (z_igae, z_igae_adj, z_hat, z_hat_adj)


# speedup vs baseline: 1.2416x; 1.0100x over previous
import jax
import jax.numpy as jnp
from jax.experimental import pallas as pl
from jax.experimental.pallas import tpu as pltpu


_N = 4096
_F = 128
_BM = 512
_NB = _N // _BM


def _layer1_kernel(adj_ref, x_ref, w1_ref, out_ref, adj16_ref, sup_ref):
    @pl.when(pl.program_id(0) == 0)
    def _():
        sup_ref[...] = jnp.tanh(x_ref[...] @ w1_ref[...]).astype(jnp.bfloat16)

    a = adj_ref[...].astype(jnp.bfloat16)
    adj16_ref[...] = a
    out_ref[...] = jax.lax.dot_general(
        a, sup_ref[...], (((1,), (0,)), ((), ())),
        preferred_element_type=jnp.float32)


_BMR = 1024
_NBR = _N // _BMR


def _layers26_kernel(adj_ref, z1_ref, w2_ref, w3_ref, w4_ref, w5_ref,
                     w6_ref, zig_ref, zhat_ref, feat_ref):
    def layer(src_ref, w_ref, active, dst_ref):
        s = src_ref[...] @ w_ref[...]
        if active:
            s = jnp.tanh(s)
        sup = s.astype(jnp.bfloat16)

        def body(i, _):
            rows = pl.ds(i * _BMR, _BMR)
            dst_ref[rows, :] = jax.lax.dot_general(
                adj_ref[rows, :], sup,
                (((1,), (0,)), ((), ())),
                preferred_element_type=jnp.float32)
            return 0

        jax.lax.fori_loop(0, _NBR, body, 0)

    layer(z1_ref, w2_ref, True, feat_ref)
    layer(feat_ref, w3_ref, False, zig_ref)
    layer(zig_ref, w4_ref, True, feat_ref)
    layer(feat_ref, w5_ref, True, feat_ref)
    layer(feat_ref, w6_ref, True, zhat_ref)


def _gram_kernel(z_ref, zfull_ref, out_ref, zf_ref):
    @pl.when(pl.program_id(0) == 0)
    def _():
        zf_ref[...] = zfull_ref[...].astype(jnp.bfloat16)

    zb = z_ref[...].astype(jnp.bfloat16)
    s = jax.lax.dot_general(
        zb, zf_ref[...], (((1,), (1,)), ((), ())),
        preferred_element_type=jnp.float32)
    out_ref[...] = jax.nn.sigmoid(s)


def _gram(z, block_rows=1024):
    n, f = z.shape
    return pl.pallas_call(
        _gram_kernel,
        grid=(n // block_rows,),
        in_specs=[
            pl.BlockSpec((block_rows, f), lambda i: (i, 0)),
            pl.BlockSpec((n, f), lambda i: (0, 0)),
        ],
        out_specs=pl.BlockSpec((block_rows, n), lambda i: (i, 0)),
        out_shape=jax.ShapeDtypeStruct((n, n), jnp.float32),
        scratch_shapes=[pltpu.VMEM((n, f), jnp.bfloat16)],
    )(z, z)


def _pad_w(w):
    fin, fout = w.shape
    return jnp.pad(w, ((0, _F - fin), (0, _F - fout)))


def kernel(x, adj, W1, W2, W3, W4, W5, W6):
    nz = W3.shape[1]
    w1, w2, w3, w4, w5, w6 = (_pad_w(w) for w in (W1, W2, W3, W4, W5, W6))
    z1, adj16 = pl.pallas_call(
        _layer1_kernel,
        grid=(_NB,),
        in_specs=[
            pl.BlockSpec((_BM, _N), lambda i: (i, 0)),
            pl.BlockSpec((_N, _F), lambda i: (0, 0)),
            pl.BlockSpec((_F, _F), lambda i: (0, 0)),
        ],
        out_specs=[
            pl.BlockSpec((_BM, _F), lambda i: (i, 0)),
            pl.BlockSpec((_BM, _N), lambda i: (i, 0)),
        ],
        out_shape=[
            jax.ShapeDtypeStruct((_N, _F), jnp.float32),
            jax.ShapeDtypeStruct((_N, _N), jnp.bfloat16),
        ],
        scratch_shapes=[pltpu.VMEM((_N, _F), jnp.bfloat16)],
    )(adj, x, w1)
    zig_pad, z_hat = pl.pallas_call(
        _layers26_kernel,
        out_shape=[
            jax.ShapeDtypeStruct((_N, _F), jnp.float32),
            jax.ShapeDtypeStruct((_N, _F), jnp.float32),
        ],
        scratch_shapes=[
            pltpu.VMEM((_N, _F), jnp.float32),
        ],
    )(adj16, z1, w2, w3, w4, w5, w6)
    z_igae = zig_pad[:, :nz]
    z_igae_adj = _gram(zig_pad)
    z_hat_adj = _gram(z_hat)
    return (z_igae, z_igae_adj, z_hat, z_hat_adj)


# speedup vs baseline: 1.3571x; 1.0930x over previous
import jax
import jax.numpy as jnp
from jax.experimental import pallas as pl
from jax.experimental.pallas import tpu as pltpu


_N = 4096
_F = 128
_BMS = 256
_NBS = _N // _BMS
_BMR = 512
_NBR = _N // _BMR


def _encdec_kernel(adj_ref, x_ref, w1_ref, w2_ref, w3_ref, w4_ref, w5_ref,
                   w6_ref, zig_ref, zhat_ref, adj16_ref, feat_ref, sup_ref):
    i = pl.program_id(0)

    @pl.when(i == 0)
    def _():
        sup_ref[...] = jnp.tanh(x_ref[...] @ w1_ref[...]).astype(jnp.bfloat16)

    a = adj_ref[...].astype(jnp.bfloat16)
    rows = pl.ds(i * _BMS, _BMS)
    adj16_ref[rows, :] = a
    feat_ref[rows, :] = jax.lax.dot_general(
        a, sup_ref[...], (((1,), (0,)), ((), ())),
        preferred_element_type=jnp.float32)

    @pl.when(i == _NBS - 1)
    def _():
        def layer(src_ref, w_ref, active, dst_ref):
            s = src_ref[...] @ w_ref[...]
            if active:
                s = jnp.tanh(s)
            sup = s.astype(jnp.bfloat16)

            def body(j, _):
                r = pl.ds(j * _BMR, _BMR)
                dst_ref[r, :] = jax.lax.dot_general(
                    adj16_ref[r, :], sup,
                    (((1,), (0,)), ((), ())),
                    preferred_element_type=jnp.float32)
                return 0

            jax.lax.fori_loop(0, _NBR, body, 0)

        layer(feat_ref, w2_ref, True, feat_ref)
        layer(feat_ref, w3_ref, False, zig_ref)
        layer(zig_ref, w4_ref, True, feat_ref)
        layer(feat_ref, w5_ref, True, feat_ref)
        layer(feat_ref, w6_ref, True, zhat_ref)


def _gram_kernel(z_ref, zfull_ref, out_ref, zf_ref):
    @pl.when(pl.program_id(0) == 0)
    def _():
        zf_ref[...] = zfull_ref[...].astype(jnp.bfloat16)

    zb = z_ref[...].astype(jnp.bfloat16)
    s = jax.lax.dot_general(
        zb, zf_ref[...], (((1,), (1,)), ((), ())),
        preferred_element_type=jnp.float32)
    out_ref[...] = jax.nn.sigmoid(s)


def _gram(z, block_rows=1024):
    n, f = z.shape
    return pl.pallas_call(
        _gram_kernel,
        grid=(n // block_rows,),
        in_specs=[
            pl.BlockSpec((block_rows, f), lambda i: (i, 0)),
            pl.BlockSpec((n, f), lambda i: (0, 0)),
        ],
        out_specs=pl.BlockSpec((block_rows, n), lambda i: (i, 0)),
        out_shape=jax.ShapeDtypeStruct((n, n), jnp.float32),
        scratch_shapes=[pltpu.VMEM((n, f), jnp.bfloat16)],
    )(z, z)


def _pad_w(w):
    fin, fout = w.shape
    return jnp.pad(w, ((0, _F - fin), (0, _F - fout)))


def kernel(x, adj, W1, W2, W3, W4, W5, W6):
    nz = W3.shape[1]
    ws = [_pad_w(w) for w in (W1, W2, W3, W4, W5, W6)]
    zig_pad, z_hat = pl.pallas_call(
        _encdec_kernel,
        grid=(_NBS,),
        in_specs=[
            pl.BlockSpec((_BMS, _N), lambda i: (i, 0)),
            pl.BlockSpec((_N, _F), lambda i: (0, 0)),
        ] + [pl.BlockSpec((_F, _F), lambda i: (0, 0))] * 6,
        out_specs=[
            pl.BlockSpec((_N, _F), lambda i: (0, 0)),
            pl.BlockSpec((_N, _F), lambda i: (0, 0)),
        ],
        out_shape=[
            jax.ShapeDtypeStruct((_N, _F), jnp.float32),
            jax.ShapeDtypeStruct((_N, _F), jnp.float32),
        ],
        scratch_shapes=[
            pltpu.VMEM((_N, _N), jnp.bfloat16),
            pltpu.VMEM((_N, _F), jnp.float32),
            pltpu.VMEM((_N, _F), jnp.bfloat16),
        ],
    )(adj, x, *ws)
    z_igae = zig_pad[:, :nz]
    z_igae_adj = _gram(zig_pad)
    z_hat_adj = _gram(z_hat)
    return (z_igae, z_igae_adj, z_hat, z_hat_adj)
